# baseline (speedup 1.0000x reference)
import functools

import jax
import jax.numpy as jnp
from jax import lax
from jax.experimental import pallas as pl
from jax.experimental.pallas import tpu as pltpu
from jax.experimental.pallas import tpu_sc as plsc

N = 10000
E = 160000
D = 256
T = 3
B = 64
H = 128
NC = 2
NS = 16
TN = T * N

_CH_A = 80
_NCH_A = N // _CH_A
_NW = NC * NS


_JA = (_NCH_A + _NW - 1) // _NW


def _embed_body(table_hbm, ids_hbm, out_hbm,
                idx0, idx1, idx2, idx3, rows0, rows1, rows2, rows3,
                semi0, semi1, semi2, semi3, semg0, semg1, semg2, semg3,
                semo0, semo1, semo2, semo3):
    c = lax.axis_index("c")
    s = lax.axis_index("s")
    w = s * NC + c
    idxq = (idx0, idx1, idx2, idx3)
    rows = (rows0, rows1, rows2, rows3)
    semi = (semi0, semi1, semi2, semi3)
    semg = (semg0, semg1, semg2, semg3)
    semo = (semo0, semo1, semo2, semo3)

    def guarded(j, f):
        @pl.when(w + _NW * j < _NCH_A)
        def _():
            f((w + _NW * j) * _CH_A)

    for j in range(_JA):
        guarded(j, lambda b, j=j: pltpu.async_copy(
            ids_hbm.at[pl.ds(b, _CH_A)], idxq[j], semi[j]))
    for j in range(_JA):
        def g(b, j=j):
            pltpu.make_async_copy(ids_hbm.at[pl.ds(b, _CH_A)], idxq[j],
                                  semi[j]).wait()
            pltpu.async_copy(table_hbm.at[idxq[j]], rows[j], semg[j])
        guarded(j, g)
    for j in range(_JA):
        def o(b, j=j):
            pltpu.make_async_copy(table_hbm.at[idxq[j]], rows[j],
                                  semg[j]).wait()
            pltpu.async_copy(rows[j], out_hbm.at[pl.ds(b, _CH_A)], semo[j])
        guarded(j, o)
    for j in range(_JA):
        guarded(j, lambda b, j=j: pltpu.make_async_copy(
            rows[j], out_hbm.at[pl.ds(b, _CH_A)], semo[j]).wait())


def _make_embed(interpret=False):
    return pl.kernel(
        _embed_body,
        out_type=jax.ShapeDtypeStruct((N, D), jnp.float32),
        mesh=plsc.VectorSubcoreMesh(core_axis_name="c", subcore_axis_name="s",
                                    num_cores=NC, num_subcores=NS),
        scratch_types=(
            [pltpu.VMEM((_CH_A,), jnp.int32) for _ in range(_JA)]
            + [pltpu.VMEM((_CH_A, D), jnp.float32) for _ in range(_JA)]
            + [pltpu.SemaphoreType.DMA for _ in range(3 * _JA)]
        ),
        interpret=interpret,
    )


def _msg_body(x_ref, w_ref, out_ref):
    out_ref[...] = jnp.dot(x_ref[...], w_ref[0],
                           preferred_element_type=jnp.float32)


def _make_msg(interpret=False):
    return pl.pallas_call(
        _msg_body,
        grid=(NC, T),
        in_specs=[
            pl.BlockSpec((N, D), lambda c, t: (0, 0)),
            pl.BlockSpec((1, D, H), lambda c, t: (t, 0, c)),
        ],
        out_specs=pl.BlockSpec((N, H), lambda c, t: (c * T + t, 0)),
        out_shape=jax.ShapeDtypeStruct((2 * TN, H), jnp.float32),
        interpret=interpret,
    )


_CH_E = 80
_EPS = E // NS
_NCH_E = _EPS // _CH_E
_NPAIR = (_NCH_E + 1) // 2
_RPS_A = 624
_RPS_B = N - 15 * _RPS_A


def _edge_body(h_hbm, ridx_hbm, dst_hbm, zeros_hbm, out_hbm,
               agg_sh, dst_v, idx0, idx1, idx2, rows0, rows1, rows2,
               semg0, semg1, semg2, sems0, sems1, sems2,
               semi0, semi1, semi2):
    c = lax.axis_index("c")
    s = lax.axis_index("s")

    @pl.when(s < NS - 1)
    def _():
        pltpu.sync_copy(zeros_hbm.at[pl.ds(0, _RPS_A)],
                        agg_sh.at[pl.ds(s * _RPS_A, _RPS_A)])

    @pl.when(s == NS - 1)
    def _():
        pltpu.sync_copy(zeros_hbm, agg_sh.at[pl.ds(15 * _RPS_A, _RPS_B)])

    pltpu.sync_copy(dst_hbm.at[s], dst_v)

    idxq = (idx0, idx1, idx2)
    rows = (rows0, rows1, rows2)
    semg = (semg0, semg1, semg2)
    sems = (sems0, sems1, sems2)
    semi = (semi0, semi1, semi2)
    ebase = c * E + s * _EPS

    def start_idx(i, b):
        pltpu.async_copy(ridx_hbm.at[pl.ds(ebase + i * _CH_E, _CH_E)],
                         idxq[b], semi[b])

    def wait_idx(i, b):
        pltpu.make_async_copy(ridx_hbm.at[pl.ds(ebase + i * _CH_E, _CH_E)],
                              idxq[b], semi[b]).wait()

    def start_gather(i, b):
        pltpu.async_copy(h_hbm.at[idxq[b]], rows[b], semg[b])

    def wait_gather(i, b):
        pltpu.make_async_copy(h_hbm.at[idxq[b]], rows[b], semg[b]).wait()

    def start_scatter(i, b):
        pltpu.async_copy(rows[b], agg_sh.at[dst_v.at[i]], sems[b], add=True)

    def wait_scatter(i, b):
        pltpu.make_async_copy(rows[b], agg_sh.at[dst_v.at[i]], sems[b]).wait()

    start_idx(0, 0)
    start_idx(1, 1)
    start_idx(2, 2)

    plsc.subcore_barrier()

    wait_idx(0, 0)
    start_gather(0, 0)
    wait_idx(1, 1)
    start_gather(1, 1)

    def tri(p, carry):
        for u in range(3):
            j = 3 * p + u

            @pl.when(j < _NCH_E)
            def _():
                wait_gather(j, u)
                start_scatter(j, u)

                @pl.when(j >= 1)
                def _():
                    wait_scatter(j - 1, (u + 2) % 3)

                @pl.when(j + 2 < _NCH_E)
                def _():
                    wait_idx(j + 2, (u + 2) % 3)
                    start_gather(j + 2, (u + 2) % 3)

                @pl.when(j + 3 < _NCH_E)
                def _():
                    start_idx(j + 3, u)

        return carry

    lax.fori_loop(0, (_NCH_E + 2) // 3, tri, 0)

    wait_scatter(_NCH_E - 1, (_NCH_E - 1) % 3)

    plsc.subcore_barrier()

    @pl.when(s < NS - 1)
    def _():
        pltpu.sync_copy(agg_sh.at[pl.ds(s * _RPS_A, _RPS_A)],
                        out_hbm.at[pl.ds(c * N + s * _RPS_A, _RPS_A)])

    @pl.when(s == NS - 1)
    def _():
        pltpu.sync_copy(agg_sh.at[pl.ds(15 * _RPS_A, _RPS_B)],
                        out_hbm.at[pl.ds(c * N + 15 * _RPS_A, _RPS_B)])


def _make_edge(interpret=False):
    return pl.kernel(
        _edge_body,
        out_type=jax.ShapeDtypeStruct((2 * N, H), jnp.float32),
        mesh=plsc.VectorSubcoreMesh(core_axis_name="c", subcore_axis_name="s",
                                    num_cores=NC, num_subcores=NS),
        scratch_types=[
            pltpu.VMEM_SHARED((N, H), jnp.float32),
            pltpu.VMEM((_NCH_E, _CH_E), jnp.int32),
            pltpu.VMEM((_CH_E,), jnp.int32),
            pltpu.VMEM((_CH_E,), jnp.int32),
            pltpu.VMEM((_CH_E,), jnp.int32),
            pltpu.VMEM((_CH_E, H), jnp.float32),
            pltpu.VMEM((_CH_E, H), jnp.float32),
            pltpu.VMEM((_CH_E, H), jnp.float32),
            pltpu.SemaphoreType.DMA,
            pltpu.SemaphoreType.DMA,
            pltpu.SemaphoreType.DMA,
            pltpu.SemaphoreType.DMA,
            pltpu.SemaphoreType.DMA,
            pltpu.SemaphoreType.DMA,
            pltpu.SemaphoreType.DMA,
            pltpu.SemaphoreType.DMA,
            pltpu.SemaphoreType.DMA,
        ],
        interpret=interpret,
    )


def _gru_body(lo_ref, hi_ref, x_ref,
              wzlo, wzhi, uz, bz, wrlo, wrhi, ur, br,
              whlo, whhi, uh, bh, w1, w2,
              r1_ref, r2_ref):
    lo = lo_ref[...]
    hi = hi_ref[...]
    x = x_ref[...]

    def mm(a, w):
        return jnp.dot(a, w[...], preferred_element_type=jnp.float32)

    za = mm(lo, wzlo) + mm(hi, wzhi) + mm(x, uz) + bz[...]
    ra = mm(lo, wrlo) + mm(hi, wrhi) + mm(x, ur) + br[...]
    z = jax.nn.sigmoid(za)
    r = jax.nn.sigmoid(ra)
    ha = mm(lo, whlo) + mm(hi, whhi) + mm(r * x, uh) + bh[...]
    hh = jnp.tanh(ha)
    rep = (1.0 - z) * x + z * hh
    r1_ref[...] = jnp.sum(rep * w1[...], axis=1).reshape(1, 1, _BND)
    r2_ref[...] = jnp.sum(rep * w2[...], axis=1).reshape(1, 1, _BND)


_BND = 2000
_NBD = N // _BND


def _make_gru(interpret=False):
    full = lambda i: (0, 0)
    return pl.pallas_call(
        _gru_body,
        grid=(_NBD,),
        in_specs=[
            pl.BlockSpec((_BND, H), lambda i: (i, 0)),
            pl.BlockSpec((_BND, H), lambda i: (_NBD + i, 0)),
            pl.BlockSpec((_BND, D), lambda i: (i, 0)),
            pl.BlockSpec((H, D), full), pl.BlockSpec((H, D), full),
            pl.BlockSpec((D, D), full), pl.BlockSpec((1, D), full),
            pl.BlockSpec((H, D), full), pl.BlockSpec((H, D), full),
            pl.BlockSpec((D, D), full), pl.BlockSpec((1, D), full),
            pl.BlockSpec((H, D), full), pl.BlockSpec((H, D), full),
            pl.BlockSpec((D, D), full), pl.BlockSpec((1, D), full),
            pl.BlockSpec((1, D), full), pl.BlockSpec((1, D), full),
        ],
        out_specs=[
            pl.BlockSpec((1, 1, _BND), lambda i: (i, 0, 0)),
            pl.BlockSpec((1, 1, _BND), lambda i: (i, 0, 0)),
        ],
        out_shape=[
            jax.ShapeDtypeStruct((_NBD, 1, _BND), jnp.float32),
            jax.ShapeDtypeStruct((_NBD, 1, _BND), jnp.float32),
        ],
        interpret=interpret,
    )


def _pool_body(gid_ref, r1_ref, r2_ref, out_ref):
    gid = gid_ref[...]
    r1 = r1_ref[...]
    r2 = r2_ref[...]
    gids = jnp.broadcast_to(gid, (B, N))
    gvals = lax.broadcasted_iota(jnp.int32, (B, N), 0)
    mask = (gids == gvals).astype(jnp.float32)
    sum2 = jnp.sum(mask * r2, axis=1, keepdims=True)
    cnt = jnp.sum(mask, axis=1, keepdims=True)
    sg = sum2 / jnp.maximum(cnt, 1.0)
    snode = jnp.sum(mask * sg, axis=0, keepdims=True)
    out_ref[...] = r1 + snode


def _make_pool(interpret=False):
    return pl.pallas_call(
        _pool_body,
        in_specs=[
            pl.BlockSpec((1, N), lambda: (0, 0)),
            pl.BlockSpec((1, N), lambda: (0, 0)),
            pl.BlockSpec((1, N), lambda: (0, 0)),
        ],
        out_specs=pl.BlockSpec((1, N), lambda: (0, 0)),
        out_shape=jax.ShapeDtypeStruct((1, N), jnp.float32),
        interpret=interpret,
    )


_make_embed = functools.cache(_make_embed)
_make_msg = functools.cache(_make_msg)
_make_edge = functools.cache(_make_edge)
_make_gru = functools.cache(_make_gru)
_make_pool = functools.cache(_make_pool)


def kernel(node_ids, edge_index, edge_types, graph_ids, embed_table, W_msg,
           Wz, Uz, bz, Wr, Ur, br, Wh, Uh, bh, W_cls):
    _embed_call = _make_embed()
    _msg_call = _make_msg()
    _edge_call = _make_edge()
    _gru_call = _make_gru()
    _pool_call = _make_pool()
    ids = node_ids.astype(jnp.int32)
    src = edge_index[0].astype(jnp.int32)
    dst = edge_index[1].astype(jnp.int32)
    et = edge_types.astype(jnp.int32)
    gid = graph_ids.astype(jnp.int32)

    ridx = et * N + src
    ridx2 = jnp.concatenate([ridx, ridx + TN])
    dst3 = dst.reshape(NS, _NCH_E, _CH_E)
    zeros = jnp.zeros((_RPS_B, H), jnp.float32)

    x = _embed_call(embed_table, ids)
    h_cat = _msg_call(x, W_msg)
    agg = _edge_call(h_cat, ridx2, dst3, zeros)

    w1 = W_cls[:D, 0].reshape(1, D)
    w2 = W_cls[D:, 0].reshape(1, D)
    r1, r2 = _gru_call(
        agg, agg, x,
        Wz[:H], Wz[H:], Uz, bz.reshape(1, D),
        Wr[:H], Wr[H:], Ur, br.reshape(1, D),
        Wh[:H], Wh[H:], Uh, bh.reshape(1, D),
        w1, w2,
    )

    logits = _pool_call(gid.reshape(1, N), r1.reshape(1, N), r2.reshape(1, N))
    return logits[0]


# speedup vs baseline: 1.0439x; 1.0439x over previous
import functools

import jax
import jax.numpy as jnp
from jax import lax
from jax.experimental import pallas as pl
from jax.experimental.pallas import tpu as pltpu
from jax.experimental.pallas import tpu_sc as plsc

N = 10000
E = 160000
D = 256
T = 3
B = 64
H = 128
NC = 2
NS = 16
TN = T * N

_CH_A = 80
_NCH_A = N // _CH_A
_NW = NC * NS


_JA = (_NCH_A + _NW - 1) // _NW


def _embed_body(table_hbm, ids_hbm, out_hbm,
                idx0, idx1, idx2, idx3, rows0, rows1, rows2, rows3,
                semi0, semi1, semi2, semi3, semg0, semg1, semg2, semg3,
                semo0, semo1, semo2, semo3):
    c = lax.axis_index("c")
    s = lax.axis_index("s")
    w = s * NC + c
    idxq = (idx0, idx1, idx2, idx3)
    rows = (rows0, rows1, rows2, rows3)
    semi = (semi0, semi1, semi2, semi3)
    semg = (semg0, semg1, semg2, semg3)
    semo = (semo0, semo1, semo2, semo3)

    def guarded(j, f):
        @pl.when(w + _NW * j < _NCH_A)
        def _():
            f((w + _NW * j) * _CH_A)

    for j in range(_JA):
        guarded(j, lambda b, j=j: pltpu.async_copy(
            ids_hbm.at[pl.ds(b, _CH_A)], idxq[j], semi[j]))
    for j in range(_JA):
        def g(b, j=j):
            pltpu.make_async_copy(ids_hbm.at[pl.ds(b, _CH_A)], idxq[j],
                                  semi[j]).wait()
            pltpu.async_copy(table_hbm.at[idxq[j]], rows[j], semg[j])
        guarded(j, g)
    for j in range(_JA):
        def o(b, j=j):
            pltpu.make_async_copy(table_hbm.at[idxq[j]], rows[j],
                                  semg[j]).wait()
            pltpu.async_copy(rows[j], out_hbm.at[pl.ds(b, _CH_A)], semo[j])
        guarded(j, o)
    for j in range(_JA):
        guarded(j, lambda b, j=j: pltpu.make_async_copy(
            rows[j], out_hbm.at[pl.ds(b, _CH_A)], semo[j]).wait())


def _make_embed(interpret=False):
    return pl.kernel(
        _embed_body,
        out_type=jax.ShapeDtypeStruct((N, D), jnp.float32),
        mesh=plsc.VectorSubcoreMesh(core_axis_name="c", subcore_axis_name="s",
                                    num_cores=NC, num_subcores=NS),
        scratch_types=(
            [pltpu.VMEM((_CH_A,), jnp.int32) for _ in range(_JA)]
            + [pltpu.VMEM((_CH_A, D), jnp.float32) for _ in range(_JA)]
            + [pltpu.SemaphoreType.DMA for _ in range(3 * _JA)]
        ),
        interpret=interpret,
    )


def _msg_body(x_ref, w_ref, out_ref):
    out_ref[...] = jnp.dot(x_ref[...], w_ref[0],
                           preferred_element_type=jnp.float32)


def _make_msg(interpret=False):
    return pl.pallas_call(
        _msg_body,
        grid=(NC, T),
        in_specs=[
            pl.BlockSpec((N, D), lambda c, t: (0, 0)),
            pl.BlockSpec((1, D, H), lambda c, t: (t, 0, c)),
        ],
        out_specs=pl.BlockSpec((N, H), lambda c, t: (c * T + t, 0)),
        out_shape=jax.ShapeDtypeStruct((2 * TN, H), jnp.float32),
        interpret=interpret,
    )


_CH_E = 80
_EPS = E // NS
_NCH_E = _EPS // _CH_E
_NPAIR = (_NCH_E + 1) // 2
_RPS_A = 624
_RPS_B = N - 15 * _RPS_A


def _edge_body(h_hbm, ridx_hbm, dst_hbm, zeros_hbm, out_hbm,
               agg_sh, dst_v, idx0, idx1, idx2, rows0, rows1, rows2,
               semg0, semg1, semg2, sems0, sems1, sems2,
               semi0, semi1, semi2):
    c = lax.axis_index("c")
    s = lax.axis_index("s")

    @pl.when(s < NS - 1)
    def _():
        pltpu.sync_copy(zeros_hbm.at[pl.ds(0, _RPS_A)],
                        agg_sh.at[pl.ds(s * _RPS_A, _RPS_A)])

    @pl.when(s == NS - 1)
    def _():
        pltpu.sync_copy(zeros_hbm, agg_sh.at[pl.ds(15 * _RPS_A, _RPS_B)])

    pltpu.sync_copy(dst_hbm.at[s], dst_v)

    idxq = (idx0, idx1, idx2)
    rows = (rows0, rows1, rows2)
    semg = (semg0, semg1, semg2)
    sems = (sems0, sems1, sems2)
    semi = (semi0, semi1, semi2)
    ebase = c * E + s * _EPS

    def start_idx(i, b):
        pltpu.async_copy(ridx_hbm.at[pl.ds(ebase + i * _CH_E, _CH_E)],
                         idxq[b], semi[b])

    def wait_idx(i, b):
        pltpu.make_async_copy(ridx_hbm.at[pl.ds(ebase + i * _CH_E, _CH_E)],
                              idxq[b], semi[b]).wait()

    def start_gather(i, b):
        pltpu.async_copy(h_hbm.at[idxq[b]], rows[b], semg[b])

    def wait_gather(i, b):
        pltpu.make_async_copy(h_hbm.at[idxq[b]], rows[b], semg[b]).wait()

    def start_scatter(i, b):
        pltpu.async_copy(rows[b], agg_sh.at[dst_v.at[i]], sems[b], add=True)

    def wait_scatter(i, b):
        pltpu.make_async_copy(rows[b], agg_sh.at[dst_v.at[i]], sems[b]).wait()

    start_idx(0, 0)
    start_idx(1, 1)
    start_idx(2, 2)

    plsc.subcore_barrier()

    wait_idx(0, 0)
    start_gather(0, 0)
    wait_idx(1, 1)
    start_gather(1, 1)

    def tri(p, carry):
        for u in range(3):
            j = 3 * p + u

            @pl.when(j < _NCH_E)
            def _():
                wait_gather(j, u)
                start_scatter(j, u)

                @pl.when(j >= 1)
                def _():
                    wait_scatter(j - 1, (u + 2) % 3)

                @pl.when(j + 2 < _NCH_E)
                def _():
                    wait_idx(j + 2, (u + 2) % 3)
                    start_gather(j + 2, (u + 2) % 3)

                @pl.when(j + 3 < _NCH_E)
                def _():
                    start_idx(j + 3, u)

        return carry

    lax.fori_loop(0, (_NCH_E + 2) // 3, tri, 0)

    wait_scatter(_NCH_E - 1, (_NCH_E - 1) % 3)

    plsc.subcore_barrier()

    @pl.when(s < NS - 1)
    def _():
        pltpu.sync_copy(agg_sh.at[pl.ds(s * _RPS_A, _RPS_A)],
                        out_hbm.at[pl.ds(c * N + s * _RPS_A, _RPS_A)])

    @pl.when(s == NS - 1)
    def _():
        pltpu.sync_copy(agg_sh.at[pl.ds(15 * _RPS_A, _RPS_B)],
                        out_hbm.at[pl.ds(c * N + 15 * _RPS_A, _RPS_B)])


def _make_edge(interpret=False):
    return pl.kernel(
        _edge_body,
        out_type=jax.ShapeDtypeStruct((2 * N, H), jnp.float32),
        mesh=plsc.VectorSubcoreMesh(core_axis_name="c", subcore_axis_name="s",
                                    num_cores=NC, num_subcores=NS),
        scratch_types=[
            pltpu.VMEM_SHARED((N, H), jnp.float32),
            pltpu.VMEM((_NCH_E, _CH_E), jnp.int32),
            pltpu.VMEM((_CH_E,), jnp.int32),
            pltpu.VMEM((_CH_E,), jnp.int32),
            pltpu.VMEM((_CH_E,), jnp.int32),
            pltpu.VMEM((_CH_E, H), jnp.float32),
            pltpu.VMEM((_CH_E, H), jnp.float32),
            pltpu.VMEM((_CH_E, H), jnp.float32),
            pltpu.SemaphoreType.DMA,
            pltpu.SemaphoreType.DMA,
            pltpu.SemaphoreType.DMA,
            pltpu.SemaphoreType.DMA,
            pltpu.SemaphoreType.DMA,
            pltpu.SemaphoreType.DMA,
            pltpu.SemaphoreType.DMA,
            pltpu.SemaphoreType.DMA,
            pltpu.SemaphoreType.DMA,
        ],
        interpret=interpret,
    )


def _gru_body(lo_ref, hi_ref, x_ref,
              wzlo, wzhi, uz, bz, wrlo, wrhi, ur, br,
              whlo, whhi, uh, bh, w1, w2,
              r1_ref, r2_ref):
    lo = lo_ref[...]
    hi = hi_ref[...]
    x = x_ref[...]

    def mm(a, w):
        return jnp.dot(a, w[...], preferred_element_type=jnp.float32)

    za = mm(lo, wzlo) + mm(hi, wzhi) + mm(x, uz) + bz[...]
    ra = mm(lo, wrlo) + mm(hi, wrhi) + mm(x, ur) + br[...]
    z = jax.nn.sigmoid(za)
    r = jax.nn.sigmoid(ra)
    ha = mm(lo, whlo) + mm(hi, whhi) + mm(r * x, uh) + bh[...]
    hh = jnp.tanh(ha)
    rep = (1.0 - z) * x + z * hh
    r1_ref[...] = jnp.sum(rep * w1[...], axis=1, keepdims=True)
    r2_ref[...] = jnp.sum(rep * w2[...], axis=1, keepdims=True)


_BND = 2000
_NBD = N // _BND


def _make_gru(interpret=False):
    full = lambda i: (0, 0)
    return pl.pallas_call(
        _gru_body,
        grid=(_NBD,),
        in_specs=[
            pl.BlockSpec((_BND, H), lambda i: (i, 0)),
            pl.BlockSpec((_BND, H), lambda i: (_NBD + i, 0)),
            pl.BlockSpec((_BND, D), lambda i: (i, 0)),
            pl.BlockSpec((H, D), full), pl.BlockSpec((H, D), full),
            pl.BlockSpec((D, D), full), pl.BlockSpec((1, D), full),
            pl.BlockSpec((H, D), full), pl.BlockSpec((H, D), full),
            pl.BlockSpec((D, D), full), pl.BlockSpec((1, D), full),
            pl.BlockSpec((H, D), full), pl.BlockSpec((H, D), full),
            pl.BlockSpec((D, D), full), pl.BlockSpec((1, D), full),
            pl.BlockSpec((1, D), full), pl.BlockSpec((1, D), full),
        ],
        out_specs=[
            pl.BlockSpec((_BND, 1), lambda i: (i, 0)),
            pl.BlockSpec((_BND, 1), lambda i: (i, 0)),
        ],
        out_shape=[
            jax.ShapeDtypeStruct((N, 1), jnp.float32),
            jax.ShapeDtypeStruct((N, 1), jnp.float32),
        ],
        interpret=interpret,
    )


def _pool_body(gid_ref, r1_ref, r2_ref, out_ref):
    gid = gid_ref[...]
    r1 = r1_ref[...]
    r2 = r2_ref[...]
    gids = jnp.broadcast_to(gid, (B, N))
    gvals = lax.broadcasted_iota(jnp.int32, (B, N), 0)
    mask = (gids == gvals).astype(jnp.float32)
    sum2 = jnp.sum(mask * r2, axis=1, keepdims=True)
    cnt = jnp.sum(mask, axis=1, keepdims=True)
    sg = sum2 / jnp.maximum(cnt, 1.0)
    snode = jnp.sum(mask * sg, axis=0, keepdims=True)
    out_ref[...] = r1 + snode


def _make_pool(interpret=False):
    return pl.pallas_call(
        _pool_body,
        in_specs=[
            pl.BlockSpec((1, N), lambda: (0, 0)),
            pl.BlockSpec((1, N), lambda: (0, 0)),
            pl.BlockSpec((1, N), lambda: (0, 0)),
        ],
        out_specs=pl.BlockSpec((1, N), lambda: (0, 0)),
        out_shape=jax.ShapeDtypeStruct((1, N), jnp.float32),
        interpret=interpret,
    )


_make_embed = functools.cache(_make_embed)
_make_msg = functools.cache(_make_msg)
_make_edge = functools.cache(_make_edge)
_make_gru = functools.cache(_make_gru)
_make_pool = functools.cache(_make_pool)


def kernel(node_ids, edge_index, edge_types, graph_ids, embed_table, W_msg,
           Wz, Uz, bz, Wr, Ur, br, Wh, Uh, bh, W_cls):
    _embed_call = _make_embed()
    _msg_call = _make_msg()
    _edge_call = _make_edge()
    _gru_call = _make_gru()
    _pool_call = _make_pool()
    ids = node_ids.astype(jnp.int32)
    src = edge_index[0].astype(jnp.int32)
    dst = edge_index[1].astype(jnp.int32)
    et = edge_types.astype(jnp.int32)
    gid = graph_ids.astype(jnp.int32)

    ridx = et * N + src
    ridx2 = jnp.concatenate([ridx, ridx + TN])
    dst3 = dst.reshape(NS, _NCH_E, _CH_E)
    zeros = jnp.zeros((_RPS_B, H), jnp.float32)

    x = _embed_call(embed_table, ids)
    h_cat = _msg_call(x, W_msg)
    agg = _edge_call(h_cat, ridx2, dst3, zeros)

    w1 = W_cls[:D, 0].reshape(1, D)
    w2 = W_cls[D:, 0].reshape(1, D)
    r1, r2 = _gru_call(
        agg, agg, x,
        Wz[:H], Wz[H:], Uz, bz.reshape(1, D),
        Wr[:H], Wr[H:], Ur, br.reshape(1, D),
        Wh[:H], Wh[H:], Uh, bh.reshape(1, D),
        w1, w2,
    )

    logits = _pool_call(gid.reshape(1, N), r1.reshape(1, N), r2.reshape(1, N))
    return logits[0]


# speedup vs baseline: 1.0522x; 1.0080x over previous
import functools

import jax
import jax.numpy as jnp
from jax import lax
from jax.experimental import pallas as pl
from jax.experimental.pallas import tpu as pltpu
from jax.experimental.pallas import tpu_sc as plsc

N = 10000
E = 160000
D = 256
T = 3
B = 64
H = 128
NC = 2
NS = 16
TN = T * N

_CH_A = 80
_NCH_A = N // _CH_A
_NW = NC * NS


_JA = (_NCH_A + _NW - 1) // _NW


def _embed_body(table_hbm, ids_hbm, out_hbm,
                idx0, idx1, idx2, idx3, rows0, rows1, rows2, rows3,
                semi0, semi1, semi2, semi3, semg0, semg1, semg2, semg3,
                semo0, semo1, semo2, semo3):
    c = lax.axis_index("c")
    s = lax.axis_index("s")
    w = s * NC + c
    idxq = (idx0, idx1, idx2, idx3)
    rows = (rows0, rows1, rows2, rows3)
    semi = (semi0, semi1, semi2, semi3)
    semg = (semg0, semg1, semg2, semg3)
    semo = (semo0, semo1, semo2, semo3)

    def guarded(j, f):
        @pl.when(w + _NW * j < _NCH_A)
        def _():
            f((w + _NW * j) * _CH_A)

    for j in range(_JA):
        guarded(j, lambda b, j=j: pltpu.async_copy(
            ids_hbm.at[pl.ds(b, _CH_A)], idxq[j], semi[j]))
    for j in range(_JA):
        def g(b, j=j):
            pltpu.make_async_copy(ids_hbm.at[pl.ds(b, _CH_A)], idxq[j],
                                  semi[j]).wait()
            pltpu.async_copy(table_hbm.at[idxq[j]], rows[j], semg[j])
        guarded(j, g)
    for j in range(_JA):
        def o(b, j=j):
            pltpu.make_async_copy(table_hbm.at[idxq[j]], rows[j],
                                  semg[j]).wait()
            pltpu.async_copy(rows[j], out_hbm.at[pl.ds(b, _CH_A)], semo[j])
        guarded(j, o)
    for j in range(_JA):
        guarded(j, lambda b, j=j: pltpu.make_async_copy(
            rows[j], out_hbm.at[pl.ds(b, _CH_A)], semo[j]).wait())


def _make_embed(interpret=False):
    return pl.kernel(
        _embed_body,
        out_type=jax.ShapeDtypeStruct((N, D), jnp.float32),
        mesh=plsc.VectorSubcoreMesh(core_axis_name="c", subcore_axis_name="s",
                                    num_cores=NC, num_subcores=NS),
        scratch_types=(
            [pltpu.VMEM((_CH_A,), jnp.int32) for _ in range(_JA)]
            + [pltpu.VMEM((_CH_A, D), jnp.float32) for _ in range(_JA)]
            + [pltpu.SemaphoreType.DMA for _ in range(3 * _JA)]
        ),
        interpret=interpret,
    )


def _msg_body(x_ref, w_ref, out_ref):
    out_ref[...] = jnp.dot(x_ref[...], w_ref[0],
                           preferred_element_type=jnp.float32)


def _make_msg(interpret=False):
    return pl.pallas_call(
        _msg_body,
        grid=(NC, T),
        in_specs=[
            pl.BlockSpec((N, D), lambda c, t: (0, 0)),
            pl.BlockSpec((1, D, H), lambda c, t: (t, 0, c)),
        ],
        out_specs=pl.BlockSpec((N, H), lambda c, t: (c * T + t, 0)),
        out_shape=jax.ShapeDtypeStruct((2 * TN, H), jnp.float32),
        interpret=interpret,
    )


_CH_E = 80
_EPS = E // NS
_NCH_E = _EPS // _CH_E
_NPAIR = (_NCH_E + 1) // 2
_RPS_A = 624
_RPS_B = N - 15 * _RPS_A


def _edge_body(h_hbm, ridx_hbm, dst_hbm, zeros_hbm, out_hbm,
               agg_sh, dst_v, idx0, idx1, idx2, rows0, rows1, rows2,
               semg0, semg1, semg2, sems0, sems1, sems2,
               semi0, semi1, semi2):
    c = lax.axis_index("c")
    s = lax.axis_index("s")

    pltpu.async_copy(dst_hbm.at[s], dst_v, sems0)

    @pl.when(s < NS - 1)
    def _():
        pltpu.async_copy(zeros_hbm.at[pl.ds(0, _RPS_A)],
                         agg_sh.at[pl.ds(s * _RPS_A, _RPS_A)], sems1)
        pltpu.make_async_copy(zeros_hbm.at[pl.ds(0, _RPS_A)],
                              agg_sh.at[pl.ds(s * _RPS_A, _RPS_A)],
                              sems1).wait()

    @pl.when(s == NS - 1)
    def _():
        pltpu.async_copy(zeros_hbm, agg_sh.at[pl.ds(15 * _RPS_A, _RPS_B)],
                         sems1)
        pltpu.make_async_copy(zeros_hbm,
                              agg_sh.at[pl.ds(15 * _RPS_A, _RPS_B)],
                              sems1).wait()

    pltpu.make_async_copy(dst_hbm.at[s], dst_v, sems0).wait()

    idxq = (idx0, idx1, idx2)
    rows = (rows0, rows1, rows2)
    semg = (semg0, semg1, semg2)
    sems = (sems0, sems1, sems2)
    semi = (semi0, semi1, semi2)
    ebase = c * E + s * _EPS

    def start_idx(i, b):
        pltpu.async_copy(ridx_hbm.at[pl.ds(ebase + i * _CH_E, _CH_E)],
                         idxq[b], semi[b])

    def wait_idx(i, b):
        pltpu.make_async_copy(ridx_hbm.at[pl.ds(ebase + i * _CH_E, _CH_E)],
                              idxq[b], semi[b]).wait()

    def start_gather(i, b):
        pltpu.async_copy(h_hbm.at[idxq[b]], rows[b], semg[b])

    def wait_gather(i, b):
        pltpu.make_async_copy(h_hbm.at[idxq[b]], rows[b], semg[b]).wait()

    def start_scatter(i, b):
        pltpu.async_copy(rows[b], agg_sh.at[dst_v.at[i]], sems[b], add=True)

    def wait_scatter(i, b):
        pltpu.make_async_copy(rows[b], agg_sh.at[dst_v.at[i]], sems[b]).wait()

    start_idx(0, 0)
    start_idx(1, 1)
    start_idx(2, 2)

    plsc.subcore_barrier()

    wait_idx(0, 0)
    start_gather(0, 0)
    wait_idx(1, 1)
    start_gather(1, 1)

    def tri(p, carry):
        for u in range(3):
            j = 3 * p + u

            @pl.when(j < _NCH_E)
            def _():
                wait_gather(j, u)
                start_scatter(j, u)

                @pl.when(j >= 1)
                def _():
                    wait_scatter(j - 1, (u + 2) % 3)

                @pl.when(j + 2 < _NCH_E)
                def _():
                    wait_idx(j + 2, (u + 2) % 3)
                    start_gather(j + 2, (u + 2) % 3)

                @pl.when(j + 3 < _NCH_E)
                def _():
                    start_idx(j + 3, u)

        return carry

    lax.fori_loop(0, (_NCH_E + 2) // 3, tri, 0)

    wait_scatter(_NCH_E - 1, (_NCH_E - 1) % 3)

    plsc.subcore_barrier()

    @pl.when(s < NS - 1)
    def _():
        pltpu.sync_copy(agg_sh.at[pl.ds(s * _RPS_A, _RPS_A)],
                        out_hbm.at[pl.ds(c * N + s * _RPS_A, _RPS_A)])

    @pl.when(s == NS - 1)
    def _():
        pltpu.sync_copy(agg_sh.at[pl.ds(15 * _RPS_A, _RPS_B)],
                        out_hbm.at[pl.ds(c * N + 15 * _RPS_A, _RPS_B)])


def _make_edge(interpret=False):
    return pl.kernel(
        _edge_body,
        out_type=jax.ShapeDtypeStruct((2 * N, H), jnp.float32),
        mesh=plsc.VectorSubcoreMesh(core_axis_name="c", subcore_axis_name="s",
                                    num_cores=NC, num_subcores=NS),
        scratch_types=[
            pltpu.VMEM_SHARED((N, H), jnp.float32),
            pltpu.VMEM((_NCH_E, _CH_E), jnp.int32),
            pltpu.VMEM((_CH_E,), jnp.int32),
            pltpu.VMEM((_CH_E,), jnp.int32),
            pltpu.VMEM((_CH_E,), jnp.int32),
            pltpu.VMEM((_CH_E, H), jnp.float32),
            pltpu.VMEM((_CH_E, H), jnp.float32),
            pltpu.VMEM((_CH_E, H), jnp.float32),
            pltpu.SemaphoreType.DMA,
            pltpu.SemaphoreType.DMA,
            pltpu.SemaphoreType.DMA,
            pltpu.SemaphoreType.DMA,
            pltpu.SemaphoreType.DMA,
            pltpu.SemaphoreType.DMA,
            pltpu.SemaphoreType.DMA,
            pltpu.SemaphoreType.DMA,
            pltpu.SemaphoreType.DMA,
        ],
        interpret=interpret,
    )


def _gru_body(lo_ref, hi_ref, x_ref,
              wzlo, wzhi, uz, bz, wrlo, wrhi, ur, br,
              whlo, whhi, uh, bh, w1, w2,
              r1_ref, r2_ref):
    lo = lo_ref[...]
    hi = hi_ref[...]
    x = x_ref[...]

    def mm(a, w):
        return jnp.dot(a, w[...], preferred_element_type=jnp.float32)

    za = mm(lo, wzlo) + mm(hi, wzhi) + mm(x, uz) + bz[...]
    ra = mm(lo, wrlo) + mm(hi, wrhi) + mm(x, ur) + br[...]
    z = jax.nn.sigmoid(za)
    r = jax.nn.sigmoid(ra)
    ha = mm(lo, whlo) + mm(hi, whhi) + mm(r * x, uh) + bh[...]
    hh = jnp.tanh(ha)
    rep = (1.0 - z) * x + z * hh
    r1_ref[...] = jnp.sum(rep * w1[...], axis=1, keepdims=True)
    r2_ref[...] = jnp.sum(rep * w2[...], axis=1, keepdims=True)


_BND = 2000
_NBD = N // _BND


def _make_gru(interpret=False):
    full = lambda i: (0, 0)
    return pl.pallas_call(
        _gru_body,
        grid=(_NBD,),
        in_specs=[
            pl.BlockSpec((_BND, H), lambda i: (i, 0)),
            pl.BlockSpec((_BND, H), lambda i: (_NBD + i, 0)),
            pl.BlockSpec((_BND, D), lambda i: (i, 0)),
            pl.BlockSpec((H, D), full), pl.BlockSpec((H, D), full),
            pl.BlockSpec((D, D), full), pl.BlockSpec((1, D), full),
            pl.BlockSpec((H, D), full), pl.BlockSpec((H, D), full),
            pl.BlockSpec((D, D), full), pl.BlockSpec((1, D), full),
            pl.BlockSpec((H, D), full), pl.BlockSpec((H, D), full),
            pl.BlockSpec((D, D), full), pl.BlockSpec((1, D), full),
            pl.BlockSpec((1, D), full), pl.BlockSpec((1, D), full),
        ],
        out_specs=[
            pl.BlockSpec((_BND, 1), lambda i: (i, 0)),
            pl.BlockSpec((_BND, 1), lambda i: (i, 0)),
        ],
        out_shape=[
            jax.ShapeDtypeStruct((N, 1), jnp.float32),
            jax.ShapeDtypeStruct((N, 1), jnp.float32),
        ],
        interpret=interpret,
    )


def _pool_body(gid_ref, r1_ref, r2_ref, out_ref):
    gid = gid_ref[...]
    r1 = r1_ref[...]
    r2 = r2_ref[...]
    gids = jnp.broadcast_to(gid, (B, N))
    gvals = lax.broadcasted_iota(jnp.int32, (B, N), 0)
    mask = (gids == gvals).astype(jnp.float32)
    sum2 = jnp.sum(mask * r2, axis=1, keepdims=True)
    cnt = jnp.sum(mask, axis=1, keepdims=True)
    sg = sum2 / jnp.maximum(cnt, 1.0)
    snode = jnp.sum(mask * sg, axis=0, keepdims=True)
    out_ref[...] = r1 + snode


def _make_pool(interpret=False):
    return pl.pallas_call(
        _pool_body,
        in_specs=[
            pl.BlockSpec((1, N), lambda: (0, 0)),
            pl.BlockSpec((1, N), lambda: (0, 0)),
            pl.BlockSpec((1, N), lambda: (0, 0)),
        ],
        out_specs=pl.BlockSpec((1, N), lambda: (0, 0)),
        out_shape=jax.ShapeDtypeStruct((1, N), jnp.float32),
        interpret=interpret,
    )


_make_embed = functools.cache(_make_embed)
_make_msg = functools.cache(_make_msg)
_make_edge = functools.cache(_make_edge)
_make_gru = functools.cache(_make_gru)
_make_pool = functools.cache(_make_pool)


def kernel(node_ids, edge_index, edge_types, graph_ids, embed_table, W_msg,
           Wz, Uz, bz, Wr, Ur, br, Wh, Uh, bh, W_cls):
    _embed_call = _make_embed()
    _msg_call = _make_msg()
    _edge_call = _make_edge()
    _gru_call = _make_gru()
    _pool_call = _make_pool()
    ids = node_ids.astype(jnp.int32)
    src = edge_index[0].astype(jnp.int32)
    dst = edge_index[1].astype(jnp.int32)
    et = edge_types.astype(jnp.int32)
    gid = graph_ids.astype(jnp.int32)

    ridx = et * N + src
    ridx2 = jnp.concatenate([ridx, ridx + TN])
    dst3 = dst.reshape(NS, _NCH_E, _CH_E)
    zeros = jnp.zeros((_RPS_B, H), jnp.float32)

    x = _embed_call(embed_table, ids)
    h_cat = _msg_call(x, W_msg)
    agg = _edge_call(h_cat, ridx2, dst3, zeros)

    w1 = W_cls[:D, 0].reshape(1, D)
    w2 = W_cls[D:, 0].reshape(1, D)
    r1, r2 = _gru_call(
        agg, agg, x,
        Wz[:H], Wz[H:], Uz, bz.reshape(1, D),
        Wr[:H], Wr[H:], Ur, br.reshape(1, D),
        Wh[:H], Wh[H:], Uh, bh.reshape(1, D),
        w1, w2,
    )

    logits = _pool_call(gid.reshape(1, N), r1.reshape(1, N), r2.reshape(1, N))
    return logits[0]
